# baseline (speedup 1.0000x reference)
import functools
import math

import jax
import jax.numpy as jnp
from jax import lax
from jax.experimental import pallas as pl
from jax.experimental.pallas import tpu as pltpu
from jax.experimental.pallas import tpu_sc as plsc

N1 = 10000
N2 = 50000
H = 128

NC = 2
NS = 16
NW = NC * NS
LB = 128

NP1 = 10240
Q2 = 12672
NQ = 4
NP2 = Q2 * NQ
AR = 12800
ZHR = 800

EDGE_SETS = (
    ("pipe", 160000, True, N1),
    ("c21", 40000, False, N1),
    ("surf", 400000, False, N2),
    ("c12", 40000, True, N2),
)


def _k_of(E):
    return 8 * math.ceil(E / (NW * LB * 8))


KMAX = max(_k_of(E) for _, E, _, _ in EDGE_SETS)


def _pad_edges(ei, E, dst_n):
    k = _k_of(E)
    ep = NW * k * LB
    trash = NP1 if dst_n == N1 else NP2
    src = jnp.concatenate([ei[0], jnp.zeros((ep - E,), jnp.int32)])
    dst = jnp.concatenate([ei[1], jnp.full((ep - E,), trash, jnp.int32)])
    return src.reshape(NW, k, LB), dst.reshape(NW, k, LB)


def _zero_acc(s, acc, zeros_hbm, dst_n):
    per = 656 if dst_n == N1 else 800
    pltpu.sync_copy(zeros_hbm.at[pl.ds(0, per)],
                    acc.at[pl.ds(s * per, per)])


def _sc_mesh():
    return plsc.VectorSubcoreMesh(core_axis_name="c", subcore_axis_name="s",
                                  num_cores=NC, num_subcores=NS)


def _remap_dst(didx, d2, j, base):
    for c16 in range(LB // 16):
        v = didx[j, pl.ds(c16 * 16, 16)] - base
        ok = (v >= 0) & (v < Q2)
        d2[0, pl.ds(c16 * 16, 16)] = jnp.where(ok, v, Q2)


def _seg_phase(name, k, dst_n, c, s, si_hbm, di_hbm, out, src_hbm,
               sidx, didx, d2, rows, zeros_hbm, acc, sem, gather=True):
    w = c * NS + s
    ng = k // 8

    def run_pass(base, remap):
        def group(g, _):
            g8 = pl.multiple_of(g * 8, 8)
            if gather:
                pltpu.sync_copy(si_hbm.at[w, pl.ds(g8, 8)], sidx)
            pltpu.sync_copy(di_hbm.at[w, pl.ds(g8, 8)], didx)
            for j in range(8):
                if gather:
                    pltpu.async_copy(src_hbm.at[sidx.at[j]], rows,
                                     sem).wait()
                if remap:
                    _remap_dst(didx, d2, j, base)
                    tgt = d2.at[0]
                else:
                    tgt = didx.at[j]
                pltpu.sync_copy(rows, acc.at[tgt], add=True)
            return ()

        lax.fori_loop(0, ng, group, ())

    if dst_n == N1:
        rpt = NP1 // NS
        _zero_acc(s, acc, zeros_hbm, dst_n)
        plsc.subcore_barrier()
        run_pass(0, False)
        plsc.subcore_barrier()
        pltpu.sync_copy(acc.at[pl.ds(s * rpt, rpt)],
                        out.at[c, pl.ds(s * rpt, rpt)])
        plsc.subcore_barrier()
    else:
        rpt = Q2 // NS
        for q in range(NQ):
            _zero_acc(s, acc, zeros_hbm, dst_n)
            plsc.subcore_barrier()
            run_pass(q * Q2, True)
            plsc.subcore_barrier()
            pltpu.sync_copy(acc.at[pl.ds(s * rpt, rpt)],
                            out.at[c, pl.ds(q * Q2 + s * rpt, rpt)])
            plsc.subcore_barrier()


def _out_rows(dst_n):
    return NP1 if dst_n == N1 else NP2


def _make_agg_kernel(ks):
    out_type = tuple(
        jax.ShapeDtypeStruct((NC, _out_rows(dst_n), H), jnp.float32)
        for _, _, _, dst_n in EDGE_SETS)

    @functools.partial(
        pl.kernel,
        out_type=out_type,
        mesh=_sc_mesh(),
        scratch_types=[
            pltpu.VMEM((8, LB), jnp.int32),
            pltpu.VMEM((8, LB), jnp.int32),
            pltpu.VMEM((1, LB), jnp.int32),
            pltpu.VMEM((LB, H), jnp.float32),
            pltpu.VMEM_SHARED((AR, H), jnp.float32),
            pltpu.SemaphoreType.DMA,
        ],
    )
    def agg(h1, h2, zeros_hbm,
            sp, dp, sc21, dc21, ss, ds2, sc12, dc12,
            o_pipe, o_c21, o_surf, o_c12,
            sidx, didx, d2, rows, acc, sem):
        c = lax.axis_index("c")
        s = lax.axis_index("s")
        srcs = {"pipe": (sp, dp), "c21": (sc21, dc21),
                "surf": (ss, ds2), "c12": (sc12, dc12)}
        outs = {"pipe": o_pipe, "c21": o_c21, "surf": o_surf, "c12": o_c12}
        for name, E, src_is_h1, dst_n in EDGE_SETS:
            si_hbm, di_hbm = srcs[name]
            _seg_phase(name, ks[name], dst_n, c, s, si_hbm, di_hbm,
                       outs[name], h1 if src_is_h1 else h2,
                       sidx, didx, d2, rows, zeros_hbm, acc, sem)

    return agg


def _make_count_kernel(ks):
    out_type = tuple(
        jax.ShapeDtypeStruct((NC, _out_rows(dst_n), H), jnp.float32)
        for _, _, _, dst_n in EDGE_SETS)

    @functools.partial(
        pl.kernel,
        out_type=out_type,
        mesh=_sc_mesh(),
        scratch_types=[
            pltpu.VMEM((8, LB), jnp.int32),
            pltpu.VMEM((1, LB), jnp.int32),
            pltpu.VMEM((LB, H), jnp.float32),
            pltpu.VMEM_SHARED((AR, H), jnp.float32),
        ],
    )
    def cnt(ones_hbm, zeros_hbm, dp, dc21, ds2, dc12,
            o_pipe, o_c21, o_surf, o_c12,
            didx, d2, ones_v, acc):
        c = lax.axis_index("c")
        s = lax.axis_index("s")
        pltpu.sync_copy(ones_hbm, ones_v)
        dsts = {"pipe": dp, "c21": dc21, "surf": ds2, "c12": dc12}
        outs = {"pipe": o_pipe, "c21": o_c21, "surf": o_surf, "c12": o_c12}
        for name, E, _, dst_n in EDGE_SETS:
            _seg_phase(name, ks[name], dst_n, c, s, None, dsts[name],
                       outs[name], None,
                       None, didx, d2, ones_v, zeros_hbm, acc, None,
                       gather=False)

    return cnt


_SQRT2 = math.sqrt(2.0)


def _gelu(x):
    return 0.5 * x * (1.0 + lax.erf(x / _SQRT2))


def _mlp2_body(x_ref, w1_ref, b1_ref, w2_ref, b2_ref, o_ref):
    t = jnp.dot(x_ref[...], w1_ref[...],
                preferred_element_type=jnp.float32) + b1_ref[...]
    t = _gelu(t)
    o_ref[...] = jnp.dot(t, w2_ref[...],
                         preferred_element_type=jnp.float32) + b2_ref[...]


def _run_enc(x, W1, b1, W2, b2, bn):
    n, din = x.shape
    return pl.pallas_call(
        _mlp2_body,
        grid=(n // bn,),
        in_specs=[
            pl.BlockSpec((bn, din), lambda i: (i, 0)),
            pl.BlockSpec((din, H), lambda i: (0, 0)),
            pl.BlockSpec((1, H), lambda i: (0, 0)),
            pl.BlockSpec((H, H), lambda i: (0, 0)),
            pl.BlockSpec((1, H), lambda i: (0, 0)),
        ],
        out_specs=pl.BlockSpec((bn, H), lambda i: (i, 0)),
        out_shape=jax.ShapeDtypeStruct((n, H), jnp.float32),
    )(x, W1, b1.reshape(1, -1), W2, b2.reshape(1, -1))


def _dec_body(x_ref, w1_ref, b1_ref, w2_ref, b2_ref, o_ref):
    t = jnp.dot(x_ref[...], w1_ref[...],
                preferred_element_type=jnp.float32) + b1_ref[...]
    t = _gelu(t)
    o_ref[...] = jnp.sum(t * w2_ref[...], axis=1, keepdims=True) + b2_ref[...]


def _run_dec(x, W1, b1, W2, b2, bn):
    n = x.shape[0]
    return pl.pallas_call(
        _dec_body,
        grid=(n // bn,),
        in_specs=[
            pl.BlockSpec((bn, H), lambda i: (i, 0)),
            pl.BlockSpec((H, H), lambda i: (0, 0)),
            pl.BlockSpec((1, H), lambda i: (0, 0)),
            pl.BlockSpec((1, H), lambda i: (0, 0)),
            pl.BlockSpec((1, 1), lambda i: (0, 0)),
        ],
        out_specs=pl.BlockSpec((bn, 1), lambda i: (i, 0)),
        out_shape=jax.ShapeDtypeStruct((n, 1), jnp.float32),
    )(x, W1, b1.reshape(1, -1), W2.reshape(1, -1), b2.reshape(1, 1))


def _dense_body(h_ref, pa_ref, pb_ref, ca_ref, cb_ref,
                wla_ref, wlb_ref, wr_ref, bias_ref, g_ref, b_ref, o_ref):
    h = h_ref[...]
    acc = jnp.dot(h, wr_ref[...], preferred_element_type=jnp.float32)
    acc = acc + bias_ref[...]
    inva = 1.0 / jnp.maximum(ca_ref[0, :, 0:1] + ca_ref[1, :, 0:1], 1.0)
    invb = 1.0 / jnp.maximum(cb_ref[0, :, 0:1] + cb_ref[1, :, 0:1], 1.0)
    ma = (pa_ref[0] + pa_ref[1]) * inva
    acc = acc + jnp.dot(ma, wla_ref[...], preferred_element_type=jnp.float32)
    mb = (pb_ref[0] + pb_ref[1]) * invb
    acc = acc + jnp.dot(mb, wlb_ref[...], preferred_element_type=jnp.float32)
    y = h + acc
    mu = jnp.mean(y, axis=-1, keepdims=True)
    d = y - mu
    var = jnp.mean(d * d, axis=-1, keepdims=True)
    o_ref[...] = d * lax.rsqrt(var + 1e-5) * g_ref[...] + b_ref[...]


def _run_dense(h, pa, pb, ca, cb, wla, wlb, wr, bias, g, b, bn):
    n = h.shape[0]
    return pl.pallas_call(
        _dense_body,
        grid=(n // bn,),
        in_specs=[
            pl.BlockSpec((bn, H), lambda i: (i, 0)),
            pl.BlockSpec((NC, bn, H), lambda i: (0, i, 0)),
            pl.BlockSpec((NC, bn, H), lambda i: (0, i, 0)),
            pl.BlockSpec((NC, bn, H), lambda i: (0, i, 0)),
            pl.BlockSpec((NC, bn, H), lambda i: (0, i, 0)),
            pl.BlockSpec((H, H), lambda i: (0, 0)),
            pl.BlockSpec((H, H), lambda i: (0, 0)),
            pl.BlockSpec((H, H), lambda i: (0, 0)),
            pl.BlockSpec((1, H), lambda i: (0, 0)),
            pl.BlockSpec((1, H), lambda i: (0, 0)),
            pl.BlockSpec((1, H), lambda i: (0, 0)),
        ],
        out_specs=pl.BlockSpec((bn, H), lambda i: (i, 0)),
        out_shape=jax.ShapeDtypeStruct((n, H), jnp.float32),
    )(h, pa, pb, ca, cb, wla, wlb, wr, bias.reshape(1, -1),
      g.reshape(1, -1), b.reshape(1, -1))


def kernel(x_1d, x_2d, ei_pipe, ei_surf, ei_c12, ei_c21, params):
    ks = {name: _k_of(E) for name, E, _, _ in EDGE_SETS}
    agg = _make_agg_kernel(ks)
    cntk = _make_count_kernel(ks)

    sp, dp = _pad_edges(ei_pipe, 160000, N1)
    sc21, dc21 = _pad_edges(ei_c21, 40000, N1)
    ss, ds2 = _pad_edges(ei_surf, 400000, N2)
    sc12, dc12 = _pad_edges(ei_c12, 40000, N2)

    zeros = jnp.zeros((ZHR, H), jnp.float32)
    ones = jnp.ones((LB, H), jnp.float32)

    c_pipe, c_c21, c_surf, c_c12 = cntk(ones, zeros, dp, dc21, ds2, dc12)

    h1 = _run_enc(x_1d, *params["enc_1d"][0], *params["enc_1d"][1], bn=1000)
    h2 = _run_enc(x_2d, *params["enc_2d"][0], *params["enc_2d"][1], bn=1000)

    for layer in params["procs"]:
        p_pipe, p_c21, p_surf, p_c12 = agg(
            h1, h2, zeros, sp, dp, sc21, dc21, ss, ds2, sc12, dc12)
        wl_p, bl_p, wr_p = layer["pipe"]
        wl_s, bl_s, wr_s = layer["surf"]
        wl_12, bl_12, wr_12 = layer["c12"]
        wl_21, bl_21, wr_21 = layer["c21"]
        h1 = _run_dense(h1, p_pipe, p_c21, c_pipe, c_c21,
                        wl_p, wl_21, wr_p + wr_21, bl_p + bl_21,
                        *layer["ln_1d"], bn=1000)
        h2 = _run_dense(h2, p_surf, p_c12, c_surf, c_c12,
                        wl_s, wl_12, wr_s + wr_12, bl_s + bl_12,
                        *layer["ln_2d"], bn=1000)

    out1 = _run_dec(h1, *params["dec_1d"][0], *params["dec_1d"][1], bn=1000)
    out2 = _run_dec(h2, *params["dec_2d"][0], *params["dec_2d"][1], bn=1000)
    return (out1, out2)


# speedup vs baseline: 2.1826x; 2.1826x over previous
import functools
import math

import jax
import jax.numpy as jnp
from jax import lax
from jax.experimental import pallas as pl
from jax.experimental.pallas import tpu as pltpu
from jax.experimental.pallas import tpu_sc as plsc

N1 = 10000
N2 = 50000
H = 128

NC = 2
NS = 16
NW = NC * NS
LB = 64

NP1 = 10240
Q2 = 12672
NQ = 4
NP2 = Q2 * NQ
AR = 12800
ZHR = 800

EDGE_SETS = (
    ("pipe", 160000, True, N1),
    ("c21", 40000, False, N1),
    ("surf", 400000, False, N2),
    ("c12", 40000, True, N2),
)


def _k_of(E):
    return 8 * math.ceil(E / (NW * LB * 8))


KMAX = max(_k_of(E) for _, E, _, _ in EDGE_SETS)


def _pad_edges(ei, E, dst_n):
    k = _k_of(E)
    ep = NW * k * LB
    trash = NP1 if dst_n == N1 else NP2
    src = jnp.concatenate([ei[0], jnp.zeros((ep - E,), jnp.int32)])
    dst = jnp.concatenate([ei[1], jnp.full((ep - E,), trash, jnp.int32)])
    return src.reshape(NW, k, LB), dst.reshape(NW, k, LB)


def _zero_acc(s, acc, zeros_hbm, dst_n):
    per = 656 if dst_n == N1 else 800
    pltpu.sync_copy(zeros_hbm.at[pl.ds(0, per)],
                    acc.at[pl.ds(s * per, per)])


def _sc_mesh():
    return plsc.VectorSubcoreMesh(core_axis_name="c", subcore_axis_name="s",
                                  num_cores=NC, num_subcores=NS)


def _remap_dst(didx, d2, j, base, trash):
    for c16 in range(LB // 16):
        v = didx[j, pl.ds(c16 * 16, 16)] - base
        ok = (v >= 0) & (v < Q2)
        d2[0, pl.ds(c16 * 16, 16)] = jnp.where(ok, v, trash)


def _seg_phase(name, k, dst_n, c, s, si_hbm, di_hbm, out, src_hbm,
               sidx, didx, d2, rows2, zeros_hbm, acc, sems, gather=True):
    w = c * NS + s
    ng = k // 8
    trash = Q2 + s * 8

    def run_pass(base, remap):
        def group(g, _):
            g8 = pl.multiple_of(g * 8, 8)
            if gather:
                pltpu.sync_copy(si_hbm.at[w, pl.ds(g8, 8)], sidx)
            pltpu.sync_copy(di_hbm.at[w, pl.ds(g8, 8)], didx)

            def scat(j):
                if remap:
                    _remap_dst(didx, d2, j, base, trash)
                    tgt = d2.at[0]
                else:
                    tgt = didx.at[j]
                pltpu.sync_copy(rows2[j % 2], acc.at[tgt], add=True)

            if gather:
                d = pltpu.async_copy(src_hbm.at[sidx.at[0]], rows2[0],
                                     sems[0])
                for j in range(1, 8):
                    dn = pltpu.async_copy(src_hbm.at[sidx.at[j]],
                                          rows2[j % 2], sems[j % 2])
                    d.wait()
                    scat(j - 1)
                    d = dn
                d.wait()
                scat(7)
            else:
                for j in range(8):
                    scat(j)
            return ()

        lax.fori_loop(0, ng, group, ())

    if dst_n == N1:
        rpt = NP1 // NS
        _zero_acc(s, acc, zeros_hbm, dst_n)
        plsc.subcore_barrier()
        run_pass(0, False)
        plsc.subcore_barrier()
        pltpu.sync_copy(acc.at[pl.ds(s * rpt, rpt)],
                        out.at[c, pl.ds(s * rpt, rpt)])
        plsc.subcore_barrier()
    else:
        rpt = Q2 // NS
        for q in range(NQ):
            _zero_acc(s, acc, zeros_hbm, dst_n)
            plsc.subcore_barrier()
            run_pass(q * Q2, True)
            plsc.subcore_barrier()
            pltpu.sync_copy(acc.at[pl.ds(s * rpt, rpt)],
                            out.at[c, pl.ds(q * Q2 + s * rpt, rpt)])
            plsc.subcore_barrier()


def _out_rows(dst_n):
    return NP1 if dst_n == N1 else NP2


def _make_agg_kernel(ks):
    out_type = tuple(
        jax.ShapeDtypeStruct((NC, _out_rows(dst_n), H), jnp.float32)
        for _, _, _, dst_n in EDGE_SETS)

    @functools.partial(
        pl.kernel,
        out_type=out_type,
        mesh=_sc_mesh(),
        scratch_types=[
            pltpu.VMEM((8, LB), jnp.int32),
            pltpu.VMEM((8, LB), jnp.int32),
            pltpu.VMEM((1, LB), jnp.int32),
            pltpu.VMEM((LB, H), jnp.float32),
            pltpu.VMEM((LB, H), jnp.float32),
            pltpu.VMEM_SHARED((AR, H), jnp.float32),
            pltpu.SemaphoreType.DMA,
            pltpu.SemaphoreType.DMA,
        ],
    )
    def agg(h1, h2, zeros_hbm,
            sp, dp, sc21, dc21, ss, ds2, sc12, dc12,
            o_pipe, o_c21, o_surf, o_c12,
            sidx, didx, d2, rows0, rows1, acc, sem0, sem1):
        c = lax.axis_index("c")
        s = lax.axis_index("s")
        srcs = {"pipe": (sp, dp), "c21": (sc21, dc21),
                "surf": (ss, ds2), "c12": (sc12, dc12)}
        outs = {"pipe": o_pipe, "c21": o_c21, "surf": o_surf, "c12": o_c12}
        for name, E, src_is_h1, dst_n in EDGE_SETS:
            si_hbm, di_hbm = srcs[name]
            _seg_phase(name, ks[name], dst_n, c, s, si_hbm, di_hbm,
                       outs[name], h1 if src_is_h1 else h2,
                       sidx, didx, d2, (rows0, rows1), zeros_hbm, acc,
                       (sem0, sem1))

    return agg


def _make_count_kernel(ks):
    out_type = tuple(
        jax.ShapeDtypeStruct((NC, _out_rows(dst_n), H), jnp.float32)
        for _, _, _, dst_n in EDGE_SETS)

    @functools.partial(
        pl.kernel,
        out_type=out_type,
        mesh=_sc_mesh(),
        scratch_types=[
            pltpu.VMEM((8, LB), jnp.int32),
            pltpu.VMEM((1, LB), jnp.int32),
            pltpu.VMEM((LB, H), jnp.float32),
            pltpu.VMEM_SHARED((AR, H), jnp.float32),
        ],
    )
    def cnt(ones_hbm, zeros_hbm, dp, dc21, ds2, dc12,
            o_pipe, o_c21, o_surf, o_c12,
            didx, d2, ones_v, acc):
        c = lax.axis_index("c")
        s = lax.axis_index("s")
        pltpu.sync_copy(ones_hbm, ones_v)
        dsts = {"pipe": dp, "c21": dc21, "surf": ds2, "c12": dc12}
        outs = {"pipe": o_pipe, "c21": o_c21, "surf": o_surf, "c12": o_c12}
        for name, E, _, dst_n in EDGE_SETS:
            _seg_phase(name, ks[name], dst_n, c, s, None, dsts[name],
                       outs[name], None,
                       None, didx, d2, (ones_v, ones_v), zeros_hbm, acc,
                       None, gather=False)

    return cnt


_SQRT2 = math.sqrt(2.0)


def _gelu(x):
    return 0.5 * x * (1.0 + lax.erf(x / _SQRT2))


def _mlp2_body(x_ref, w1_ref, b1_ref, w2_ref, b2_ref, o_ref):
    t = jnp.dot(x_ref[...], w1_ref[...],
                preferred_element_type=jnp.float32) + b1_ref[...]
    t = _gelu(t)
    o_ref[...] = jnp.dot(t, w2_ref[...],
                         preferred_element_type=jnp.float32) + b2_ref[...]


def _run_enc(x, W1, b1, W2, b2, bn):
    n, din = x.shape
    return pl.pallas_call(
        _mlp2_body,
        grid=(n // bn,),
        in_specs=[
            pl.BlockSpec((bn, din), lambda i: (i, 0)),
            pl.BlockSpec((din, H), lambda i: (0, 0)),
            pl.BlockSpec((1, H), lambda i: (0, 0)),
            pl.BlockSpec((H, H), lambda i: (0, 0)),
            pl.BlockSpec((1, H), lambda i: (0, 0)),
        ],
        out_specs=pl.BlockSpec((bn, H), lambda i: (i, 0)),
        out_shape=jax.ShapeDtypeStruct((n, H), jnp.float32),
    )(x, W1, b1.reshape(1, -1), W2, b2.reshape(1, -1))


def _dec_body(x_ref, w1_ref, b1_ref, w2_ref, b2_ref, o_ref):
    t = jnp.dot(x_ref[...], w1_ref[...],
                preferred_element_type=jnp.float32) + b1_ref[...]
    t = _gelu(t)
    o_ref[...] = jnp.sum(t * w2_ref[...], axis=1, keepdims=True) + b2_ref[...]


def _run_dec(x, W1, b1, W2, b2, bn):
    n = x.shape[0]
    return pl.pallas_call(
        _dec_body,
        grid=(n // bn,),
        in_specs=[
            pl.BlockSpec((bn, H), lambda i: (i, 0)),
            pl.BlockSpec((H, H), lambda i: (0, 0)),
            pl.BlockSpec((1, H), lambda i: (0, 0)),
            pl.BlockSpec((1, H), lambda i: (0, 0)),
            pl.BlockSpec((1, 1), lambda i: (0, 0)),
        ],
        out_specs=pl.BlockSpec((bn, 1), lambda i: (i, 0)),
        out_shape=jax.ShapeDtypeStruct((n, 1), jnp.float32),
    )(x, W1, b1.reshape(1, -1), W2.reshape(1, -1), b2.reshape(1, 1))


def _dense_body(h_ref, pa_ref, pb_ref, ca_ref, cb_ref,
                wla_ref, wlb_ref, wr_ref, bias_ref, g_ref, b_ref, o_ref):
    h = h_ref[...]
    acc = jnp.dot(h, wr_ref[...], preferred_element_type=jnp.float32)
    acc = acc + bias_ref[...]
    inva = 1.0 / jnp.maximum(ca_ref[0, :, 0:1] + ca_ref[1, :, 0:1], 1.0)
    invb = 1.0 / jnp.maximum(cb_ref[0, :, 0:1] + cb_ref[1, :, 0:1], 1.0)
    ma = (pa_ref[0] + pa_ref[1]) * inva
    acc = acc + jnp.dot(ma, wla_ref[...], preferred_element_type=jnp.float32)
    mb = (pb_ref[0] + pb_ref[1]) * invb
    acc = acc + jnp.dot(mb, wlb_ref[...], preferred_element_type=jnp.float32)
    y = h + acc
    mu = jnp.mean(y, axis=-1, keepdims=True)
    d = y - mu
    var = jnp.mean(d * d, axis=-1, keepdims=True)
    o_ref[...] = d * lax.rsqrt(var + 1e-5) * g_ref[...] + b_ref[...]


def _run_dense(h, pa, pb, ca, cb, wla, wlb, wr, bias, g, b, bn):
    n = h.shape[0]
    return pl.pallas_call(
        _dense_body,
        grid=(n // bn,),
        in_specs=[
            pl.BlockSpec((bn, H), lambda i: (i, 0)),
            pl.BlockSpec((NC, bn, H), lambda i: (0, i, 0)),
            pl.BlockSpec((NC, bn, H), lambda i: (0, i, 0)),
            pl.BlockSpec((NC, bn, H), lambda i: (0, i, 0)),
            pl.BlockSpec((NC, bn, H), lambda i: (0, i, 0)),
            pl.BlockSpec((H, H), lambda i: (0, 0)),
            pl.BlockSpec((H, H), lambda i: (0, 0)),
            pl.BlockSpec((H, H), lambda i: (0, 0)),
            pl.BlockSpec((1, H), lambda i: (0, 0)),
            pl.BlockSpec((1, H), lambda i: (0, 0)),
            pl.BlockSpec((1, H), lambda i: (0, 0)),
        ],
        out_specs=pl.BlockSpec((bn, H), lambda i: (i, 0)),
        out_shape=jax.ShapeDtypeStruct((n, H), jnp.float32),
    )(h, pa, pb, ca, cb, wla, wlb, wr, bias.reshape(1, -1),
      g.reshape(1, -1), b.reshape(1, -1))


def kernel(x_1d, x_2d, ei_pipe, ei_surf, ei_c12, ei_c21, params):
    ks = {name: _k_of(E) for name, E, _, _ in EDGE_SETS}
    agg = _make_agg_kernel(ks)
    cntk = _make_count_kernel(ks)

    sp, dp = _pad_edges(ei_pipe, 160000, N1)
    sc21, dc21 = _pad_edges(ei_c21, 40000, N1)
    ss, ds2 = _pad_edges(ei_surf, 400000, N2)
    sc12, dc12 = _pad_edges(ei_c12, 40000, N2)

    zeros = jnp.zeros((ZHR, H), jnp.float32)
    ones = jnp.ones((LB, H), jnp.float32)

    c_pipe, c_c21, c_surf, c_c12 = cntk(ones, zeros, dp, dc21, ds2, dc12)

    h1 = _run_enc(x_1d, *params["enc_1d"][0], *params["enc_1d"][1], bn=1000)
    h2 = _run_enc(x_2d, *params["enc_2d"][0], *params["enc_2d"][1], bn=1000)

    for layer in params["procs"]:
        p_pipe, p_c21, p_surf, p_c12 = agg(
            h1, h2, zeros, sp, dp, sc21, dc21, ss, ds2, sc12, dc12)
        wl_p, bl_p, wr_p = layer["pipe"]
        wl_s, bl_s, wr_s = layer["surf"]
        wl_12, bl_12, wr_12 = layer["c12"]
        wl_21, bl_21, wr_21 = layer["c21"]
        h1 = _run_dense(h1, p_pipe, p_c21, c_pipe, c_c21,
                        wl_p, wl_21, wr_p + wr_21, bl_p + bl_21,
                        *layer["ln_1d"], bn=1000)
        h2 = _run_dense(h2, p_surf, p_c12, c_surf, c_c12,
                        wl_s, wl_12, wr_s + wr_12, bl_s + bl_12,
                        *layer["ln_2d"], bn=1000)

    out1 = _run_dec(h1, *params["dec_1d"][0], *params["dec_1d"][1], bn=1000)
    out2 = _run_dec(h2, *params["dec_2d"][0], *params["dec_2d"][1], bn=1000)
    return (out1, out2)
